# baseline (speedup 1.0000x reference)
import functools

import jax
import jax.numpy as jnp
from jax import lax
from jax.experimental import pallas as pl
from jax.experimental.pallas import tpu as pltpu
from jax.experimental.pallas import tpu_sc as plsc

NC = 2
NS = 16
NW = NC * NS
C = 128


def _sc_agg_kernel(n, d, k, n_acc):
    zr = n_acc // NS

    out_type = jax.ShapeDtypeStruct((NC, n_acc, d), jnp.float32)
    scratch = [
        pltpu.VMEM_SHARED((n_acc, d), jnp.float32),
        [pltpu.VMEM((2, C), jnp.int32) for _ in range(4)],
        [pltpu.VMEM((C, d), jnp.float32) for _ in range(2)],
        [pltpu.SemaphoreType.DMA for _ in range(2)],
        [pltpu.SemaphoreType.DMA for _ in range(4)],
        [pltpu.SemaphoreType.DMA for _ in range(4)],
    ]

    mesh = plsc.VectorSubcoreMesh(core_axis_name="c", subcore_axis_name="s",
                                  num_cores=NC, num_subcores=NS)

    def body(h_hbm, sd_hbm, zrows_hbm, agg_out, acc, ib, rb, semg, semi, sems):
        c = lax.axis_index("c")
        s = lax.axis_index("s")
        wid = c * NS + s

        pltpu.sync_copy(zrows_hbm, acc.at[pl.ds(s * zr, zr)])
        plsc.subcore_barrier()

        def idx_start(j, t):
            pltpu.async_copy(sd_hbm.at[wid, j], ib[t], semi[t])

        def idx_wait(t):
            pltpu.make_async_copy(sd_hbm.at[wid, 0], ib[t], semi[t]).wait()

        def gather_start(ti, tr):
            pltpu.async_copy(h_hbm.at[ib[ti].at[0]], rb[tr], semg[tr])

        def gather_wait(tr):
            pltpu.make_async_copy(h_hbm.at[ib[0].at[0]], rb[tr], semg[tr]).wait()

        def scatter_start(ti, tr):
            pltpu.async_copy(rb[tr], acc.at[ib[ti].at[1]], sems[ti], add=True)

        def scatter_wait(ti):
            pltpu.make_async_copy(rb[ti % 2], acc.at[ib[ti].at[1]],
                                  sems[ti]).wait()

        pltpu.sync_copy(sd_hbm.at[wid, 0], ib[0])
        gather_start(0, 0)
        idx_start(1, 1)

        def step(qq, carry):
            j0 = qq * 4
            first = qq == 0
            more = j0 + 4 < k
            for t in range(4):
                j = j0 + t
                tr = t % 2
                gather_wait(tr)
                scatter_start(t, tr)
                if t == 0:
                    @pl.when(jnp.logical_not(first))
                    def _():
                        scatter_wait(3)

                    idx_wait(1)
                    gather_start(1, 1)
                elif t == 3:
                    @pl.when(more)
                    def _():
                        scatter_wait(2)
                        idx_wait(0)
                        gather_start(0, 0)
                else:
                    scatter_wait(t - 1)
                    idx_wait(t + 1)
                    gather_start(t + 1, (t + 1) % 2)
                ti2 = (t + 2) % 4
                if t < 2:
                    idx_start(j + 2, ti2)
                else:
                    @pl.when(more)
                    def _():
                        idx_start(j + 2, ti2)
            return carry

        lax.fori_loop(0, k // 4, step, 0)
        scatter_wait(2)
        scatter_wait(3)
        plsc.subcore_barrier()

        pltpu.sync_copy(acc.at[pl.ds(s * zr, zr)], agg_out.at[c, pl.ds(s * zr, zr)])

    return pl.kernel(body, out_type=out_type, mesh=mesh,
                     scratch_types=scratch)


def _sc_deg_kernel(k, n_acc):
    rows = n_acc // 128
    out_type = jax.ShapeDtypeStruct((NC, rows, 128), jnp.float32)
    scratch = [
        pltpu.VMEM_SHARED((rows, 128), jnp.float32),
        pltpu.VMEM((rows, 128), jnp.float32),
        pltpu.VMEM((2, C), jnp.int32),
        pltpu.VMEM((2, C), jnp.int32),
        pltpu.VMEM((rows,), jnp.int32),
        pltpu.SemaphoreType.DMA,
        pltpu.SemaphoreType.DMA,
    ]
    mesh = plsc.VectorSubcoreMesh(core_axis_name="c", subcore_axis_name="s",
                                  num_cores=NC, num_subcores=NS)

    def body(sd_hbm, deg_out, dacc, hist, i0, i1, idn, semi0, semi1):
        c = lax.axis_index("c")
        s = lax.axis_index("s")
        wid = c * NS + s

        z16 = jnp.zeros((16,), jnp.float32)

        def zrow(r, carry):
            for cc in range(8):
                hist[r, pl.ds(cc * 16, 16)] = z16
            return carry

        lax.fori_loop(0, rows, zrow, 0)

        def irow(r, carry):
            idn[pl.ds(r * 16, 16)] = lax.iota(jnp.int32, 16) + r * 16
            return carry

        lax.fori_loop(0, rows // 16, irow, 0)

        @pl.when(s == 0)
        def _():
            pltpu.sync_copy(hist, dacc)
        plsc.subcore_barrier()

        def idx_start(j, buf, sem):
            pltpu.async_copy(sd_hbm.at[wid, j], buf, sem)

        def idx_wait(buf, sem):
            pltpu.make_async_copy(sd_hbm.at[wid, 0], buf, sem).wait()

        ones = jnp.ones((16,), jnp.float32)

        def chunk(ibuf):
            for r in range(C // 16):
                v = ibuf[1, pl.ds(r * 16, 16)]
                vhi = lax.shift_right_logical(v, 7)
                vlo = lax.bitwise_and(v, 127)
                plsc.addupdate_scatter(hist, [vhi, vlo], ones)

        idx_start(0, i0, semi0)

        def step(jj, carry):
            j0 = jj * 2
            more = jj + 1 < k // 2
            idx_wait(i0, semi0)
            idx_start(j0 + 1, i1, semi1)
            chunk(i0)
            idx_wait(i1, semi1)

            @pl.when(more)
            def _():
                idx_start(j0 + 2, i0, semi0)

            chunk(i1)
            return carry

        lax.fori_loop(0, k // 2, step, 0)

        pltpu.sync_copy(hist, dacc.at[idn], add=True)
        plsc.subcore_barrier()

        @pl.when(s < rows // 8)
        def _():
            pltpu.sync_copy(dacc.at[pl.ds(s * 8, 8)],
                            deg_out.at[c, pl.ds(s * 8, 8)])

    return pl.kernel(body, out_type=out_type, mesh=mesh,
                     scratch_types=scratch,
                     compiler_params=pltpu.CompilerParams(
                         needs_layout_passes=False))


def _dense_body(h_ref, agg_ref, deg_ref, ws_ref, wn_ref, b_ref, out_ref):
    deg = jnp.maximum(deg_ref[...], 1.0)
    agg = (agg_ref[0] + agg_ref[1]) / deg
    hn = jnp.dot(agg, wn_ref[...], preferred_element_type=jnp.float32)
    hs = jnp.dot(h_ref[...], ws_ref[...], preferred_element_type=jnp.float32)
    h = jnp.maximum(hs + hn + b_ref[...], 0.0)
    nrm = jnp.maximum(jnp.sqrt(jnp.sum(h * h, axis=1, keepdims=True)), 1e-12)
    out_ref[...] = h / nrm


def _dense_layer(h, agg, deg, w_self, w_neigh, bias_row, br=400):
    n, d = h.shape
    grid = (n // br,)
    return pl.pallas_call(
        _dense_body,
        grid=grid,
        in_specs=[
            pl.BlockSpec((br, d), lambda i: (i, 0)),
            pl.BlockSpec((NC, br, d), lambda i: (0, i, 0)),
            pl.BlockSpec((br, 1), lambda i: (i, 0)),
            pl.BlockSpec((d, d), lambda i: (0, 0)),
            pl.BlockSpec((d, d), lambda i: (0, 0)),
            pl.BlockSpec((1, d), lambda i: (0, 0)),
        ],
        out_specs=pl.BlockSpec((br, d), lambda i: (i, 0)),
        out_shape=jax.ShapeDtypeStruct((n, d), jnp.float32),
    )(h, agg, deg, w_self, w_neigh, bias_row)


def kernel(x, edge_index, W_self, W_neigh, bias):
    n, d = x.shape
    e = edge_index.shape[1]
    L = W_self.shape[0]

    k = -(-e // (NW * C))
    k += (-k) % 4
    e_pad = k * NW * C
    n_acc = -(-(n + 128) // 128) * 128
    npad_rows = n_acc - n

    src = edge_index[0]
    dst = edge_index[1]
    pad = e_pad - e
    if pad:
        ar = jnp.arange(pad, dtype=jnp.int32)
        src = jnp.concatenate([src, ar % n])
        dst = jnp.concatenate([dst, n + (ar % npad_rows)])
    sd = jnp.stack([src.reshape(NW, k, C), dst.reshape(NW, k, C)], axis=2)

    zrows = jnp.zeros((n_acc // NS, d), jnp.float32)

    agg_fn = _sc_agg_kernel(n, d, k, n_acc)
    deg_fn = _sc_deg_kernel(k, n_acc)

    degp = deg_fn(sd)
    deg = (degp[0] + degp[1]).reshape(n_acc)[:n, None]
    h = x
    for l in range(L):
        agg = agg_fn(h, sd, zrows)
        h = _dense_layer(h, agg, deg, W_self[l], W_neigh[l], bias[l][None, :])
    return h


# speedup vs baseline: 1.0710x; 1.0710x over previous
import functools

import jax
import jax.numpy as jnp
from jax import lax
from jax.experimental import pallas as pl
from jax.experimental.pallas import tpu as pltpu
from jax.experimental.pallas import tpu_sc as plsc

NC = 2
NS = 16
NW = NC * NS
C = 128


def _sc_agg_kernel(n, d, k, n_acc, with_deg=False):
    zr = n_acc // NS
    rows = n_acc // 128

    out_type = [jax.ShapeDtypeStruct((NC, n_acc, d), jnp.float32)]
    scratch = [
        pltpu.VMEM_SHARED((n_acc, d), jnp.float32),
        [pltpu.VMEM((2, C), jnp.int32) for _ in range(4)],
        [pltpu.VMEM((C, d), jnp.float32) for _ in range(2)],
        [pltpu.SemaphoreType.DMA for _ in range(2)],
        [pltpu.SemaphoreType.DMA for _ in range(4)],
        [pltpu.SemaphoreType.DMA for _ in range(4)],
    ]
    if with_deg:
        out_type.append(jax.ShapeDtypeStruct((NC, rows, 128), jnp.float32))
        scratch += [
            pltpu.VMEM_SHARED((rows, 128), jnp.float32),
            pltpu.VMEM((rows, 128), jnp.float32),
            pltpu.VMEM((rows,), jnp.int32),
        ]

    mesh = plsc.VectorSubcoreMesh(core_axis_name="c", subcore_axis_name="s",
                                  num_cores=NC, num_subcores=NS)

    def body(*refs):
        if with_deg:
            (h_hbm, sd_hbm, zrows_hbm, agg_out, deg_out,
             acc, ib, rb, semg, semi, sems, dacc, hist, idn) = refs
        else:
            (h_hbm, sd_hbm, zrows_hbm, agg_out,
             acc, ib, rb, semg, semi, sems) = refs
        c = lax.axis_index("c")
        s = lax.axis_index("s")
        wid = c * NS + s

        pltpu.sync_copy(zrows_hbm, acc.at[pl.ds(s * zr, zr)])
        if with_deg:
            z16 = jnp.zeros((16,), jnp.float32)

            def zrow(r, carry):
                for cc in range(8):
                    hist[r, pl.ds(cc * 16, 16)] = z16
                return carry

            lax.fori_loop(0, rows, zrow, 0)

            def irow(r, carry):
                idn[pl.ds(r * 16, 16)] = lax.iota(jnp.int32, 16) + r * 16
                return carry

            lax.fori_loop(0, rows // 16, irow, 0)

            @pl.when(s == 0)
            def _():
                pltpu.sync_copy(hist, dacc)
        plsc.subcore_barrier()

        ones = jnp.ones((16,), jnp.float32)

        def count(ibuf):
            for r in range(C // 16):
                v = ibuf[1, pl.ds(r * 16, 16)]
                vhi = lax.shift_right_logical(v, 7)
                vlo = lax.bitwise_and(v, 127)
                plsc.addupdate_scatter(hist, [vhi, vlo], ones)

        def idx_start(j, t):
            pltpu.async_copy(sd_hbm.at[wid, j], ib[t], semi[t])

        def idx_wait(t):
            pltpu.make_async_copy(sd_hbm.at[wid, 0], ib[t], semi[t]).wait()

        def gather_start(ti, tr):
            pltpu.async_copy(h_hbm.at[ib[ti].at[0]], rb[tr], semg[tr])

        def gather_wait(tr):
            pltpu.make_async_copy(h_hbm.at[ib[0].at[0]], rb[tr], semg[tr]).wait()

        def scatter_start(ti, tr):
            pltpu.async_copy(rb[tr], acc.at[ib[ti].at[1]], sems[ti], add=True)

        def scatter_wait(ti):
            pltpu.make_async_copy(rb[ti % 2], acc.at[ib[ti].at[1]],
                                  sems[ti]).wait()

        pltpu.sync_copy(sd_hbm.at[wid, 0], ib[0])
        gather_start(0, 0)
        idx_start(1, 1)

        def step(qq, carry):
            j0 = qq * 4
            first = qq == 0
            more = j0 + 4 < k
            for t in range(4):
                j = j0 + t
                tr = t % 2
                gather_wait(tr)
                scatter_start(t, tr)
                if with_deg:
                    count(ib[t])
                if t == 0:
                    @pl.when(jnp.logical_not(first))
                    def _():
                        scatter_wait(3)

                    idx_wait(1)
                    gather_start(1, 1)
                elif t == 3:
                    @pl.when(more)
                    def _():
                        scatter_wait(2)
                        idx_wait(0)
                        gather_start(0, 0)
                else:
                    scatter_wait(t - 1)
                    idx_wait(t + 1)
                    gather_start(t + 1, (t + 1) % 2)
                ti2 = (t + 2) % 4
                if t < 2:
                    idx_start(j + 2, ti2)
                else:
                    @pl.when(more)
                    def _():
                        idx_start(j + 2, ti2)
            return carry

        lax.fori_loop(0, k // 4, step, 0)
        scatter_wait(2)
        scatter_wait(3)
        if with_deg:
            pltpu.sync_copy(hist, dacc.at[idn], add=True)
        plsc.subcore_barrier()

        pltpu.sync_copy(acc.at[pl.ds(s * zr, zr)], agg_out.at[c, pl.ds(s * zr, zr)])
        if with_deg:
            @pl.when(s < rows // 8)
            def _():
                pltpu.sync_copy(dacc.at[pl.ds(s * 8, 8)],
                                deg_out.at[c, pl.ds(s * 8, 8)])

    params = pltpu.CompilerParams(needs_layout_passes=False) if with_deg else None
    return pl.kernel(body, out_type=tuple(out_type), mesh=mesh,
                     scratch_types=scratch, compiler_params=params)


def _dense_body(h_ref, agg_ref, deg_ref, ws_ref, wn_ref, b_ref, out_ref):
    deg = jnp.maximum(deg_ref[...], 1.0)
    agg = (agg_ref[0] + agg_ref[1]) / deg
    hn = jnp.dot(agg, wn_ref[...], preferred_element_type=jnp.float32)
    hs = jnp.dot(h_ref[...], ws_ref[...], preferred_element_type=jnp.float32)
    h = jnp.maximum(hs + hn + b_ref[...], 0.0)
    nrm = jnp.maximum(jnp.sqrt(jnp.sum(h * h, axis=1, keepdims=True)), 1e-12)
    out_ref[...] = h / nrm


def _dense_layer(h, agg, deg, w_self, w_neigh, bias_row, br=400):
    n, d = h.shape
    grid = (n // br,)
    return pl.pallas_call(
        _dense_body,
        grid=grid,
        in_specs=[
            pl.BlockSpec((br, d), lambda i: (i, 0)),
            pl.BlockSpec((NC, br, d), lambda i: (0, i, 0)),
            pl.BlockSpec((br, 1), lambda i: (i, 0)),
            pl.BlockSpec((d, d), lambda i: (0, 0)),
            pl.BlockSpec((d, d), lambda i: (0, 0)),
            pl.BlockSpec((1, d), lambda i: (0, 0)),
        ],
        out_specs=pl.BlockSpec((br, d), lambda i: (i, 0)),
        out_shape=jax.ShapeDtypeStruct((n, d), jnp.float32),
    )(h, agg, deg, w_self, w_neigh, bias_row)


def kernel(x, edge_index, W_self, W_neigh, bias):
    n, d = x.shape
    e = edge_index.shape[1]
    L = W_self.shape[0]

    k = -(-e // (NW * C))
    k += (-k) % 4
    e_pad = k * NW * C
    n_acc = -(-(n + 128) // 128) * 128
    npad_rows = n_acc - n

    src = edge_index[0]
    dst = edge_index[1]
    pad = e_pad - e
    if pad:
        ar = jnp.arange(pad, dtype=jnp.int32)
        src = jnp.concatenate([src, ar % n])
        dst = jnp.concatenate([dst, n + (ar % npad_rows)])
    sd = jnp.stack([src.reshape(NW, k, C), dst.reshape(NW, k, C)], axis=2)

    zrows = jnp.zeros((n_acc // NS, d), jnp.float32)

    agg_deg_fn = _sc_agg_kernel(n, d, k, n_acc, with_deg=True)
    agg_fn = _sc_agg_kernel(n, d, k, n_acc)

    agg, degp = agg_deg_fn(x, sd, zrows)
    deg = (degp[0] + degp[1]).reshape(n_acc)[:n, None]
    h = _dense_layer(x, agg, deg, W_self[0], W_neigh[0], bias[0][None, :])
    for l in range(1, L):
        (agg,) = agg_fn(h, sd, zrows)
        h = _dense_layer(h, agg, deg, W_self[l], W_neigh[l], bias[l][None, :])
    return h


# speedup vs baseline: 1.0974x; 1.0246x over previous
import functools

import jax
import jax.numpy as jnp
from jax import lax
from jax.experimental import pallas as pl
from jax.experimental.pallas import tpu as pltpu
from jax.experimental.pallas import tpu_sc as plsc

NC = 2
NS = 16
NW = NC * NS
C = 128


def _sc_agg_kernel(n, d, k, n_acc, with_deg=False):
    zr = n_acc // NS
    rows = n_acc // 128

    out_type = [jax.ShapeDtypeStruct((NC, n_acc, d), jnp.float32)]
    scratch = [
        pltpu.VMEM_SHARED((n_acc, d), jnp.float32),
        [pltpu.VMEM((2, C), jnp.int32) for _ in range(4)],
        [pltpu.VMEM((C, d), jnp.float32) for _ in range(2)],
        [pltpu.SemaphoreType.DMA for _ in range(2)],
        [pltpu.SemaphoreType.DMA for _ in range(4)],
        [pltpu.SemaphoreType.DMA for _ in range(4)],
    ]
    if with_deg:
        out_type.append(jax.ShapeDtypeStruct((NC, rows, 128), jnp.float32))
        scratch += [
            pltpu.VMEM_SHARED((rows, 128), jnp.float32),
            pltpu.VMEM((rows, 128), jnp.float32),
            pltpu.VMEM((rows,), jnp.int32),
        ]

    mesh = plsc.VectorSubcoreMesh(core_axis_name="c", subcore_axis_name="s",
                                  num_cores=NC, num_subcores=NS)

    def body(*refs):
        if with_deg:
            (h_hbm, sd_hbm, agg_out, deg_out,
             acc, ib, rb, semg, semi, sems, dacc, hist, idn) = refs
        else:
            (h_hbm, sd_hbm, agg_out,
             acc, ib, rb, semg, semi, sems) = refs
        c = lax.axis_index("c")
        s = lax.axis_index("s")
        wid = c * NS + s

        z16 = jnp.zeros((16,), jnp.float32)

        def zrb(r, carry):
            for cc in range(d // 16):
                rb[0][r, pl.ds(cc * 16, 16)] = z16
            return carry

        lax.fori_loop(0, C, zrb, 0)
        for q in range(zr // C):
            pltpu.sync_copy(rb[0], acc.at[pl.ds(s * zr + q * C, C)])
        if with_deg:
            def zrow(r, carry):
                for cc in range(8):
                    hist[r, pl.ds(cc * 16, 16)] = z16
                return carry

            lax.fori_loop(0, rows, zrow, 0)

            def irow(r, carry):
                idn[pl.ds(r * 16, 16)] = lax.iota(jnp.int32, 16) + r * 16
                return carry

            lax.fori_loop(0, rows // 16, irow, 0)

            @pl.when(s == 0)
            def _():
                pltpu.sync_copy(hist, dacc)
        plsc.subcore_barrier()

        ones = jnp.ones((16,), jnp.float32)

        def count(ibuf):
            for r in range(C // 16):
                v = ibuf[1, pl.ds(r * 16, 16)]
                vhi = lax.shift_right_logical(v, 7)
                vlo = lax.bitwise_and(v, 127)
                plsc.addupdate_scatter(hist, [vhi, vlo], ones)

        def idx_start(j, t):
            pltpu.async_copy(sd_hbm.at[wid, j], ib[t], semi[t])

        def idx_wait(t):
            pltpu.make_async_copy(sd_hbm.at[wid, 0], ib[t], semi[t]).wait()

        def gather_start(ti, tr):
            pltpu.async_copy(h_hbm.at[ib[ti].at[0]], rb[tr], semg[tr])

        def gather_wait(tr):
            pltpu.make_async_copy(h_hbm.at[ib[0].at[0]], rb[tr], semg[tr]).wait()

        def scatter_start(ti, tr):
            pltpu.async_copy(rb[tr], acc.at[ib[ti].at[1]], sems[ti], add=True)

        def scatter_wait(ti):
            pltpu.make_async_copy(rb[ti % 2], acc.at[ib[ti].at[1]],
                                  sems[ti]).wait()

        pltpu.sync_copy(sd_hbm.at[wid, 0], ib[0])
        gather_start(0, 0)
        idx_start(1, 1)

        def step(qq, carry):
            j0 = qq * 4
            first = qq == 0
            more = j0 + 4 < k
            for t in range(4):
                j = j0 + t
                tr = t % 2
                gather_wait(tr)
                scatter_start(t, tr)
                if with_deg:
                    count(ib[t])
                if t == 0:
                    @pl.when(jnp.logical_not(first))
                    def _():
                        scatter_wait(3)

                    idx_wait(1)
                    gather_start(1, 1)
                elif t == 3:
                    @pl.when(more)
                    def _():
                        scatter_wait(2)
                        idx_wait(0)
                        gather_start(0, 0)
                else:
                    scatter_wait(t - 1)
                    idx_wait(t + 1)
                    gather_start(t + 1, (t + 1) % 2)
                ti2 = (t + 2) % 4
                if t < 2:
                    idx_start(j + 2, ti2)
                else:
                    @pl.when(more)
                    def _():
                        idx_start(j + 2, ti2)
            return carry

        lax.fori_loop(0, k // 4, step, 0)
        scatter_wait(2)
        scatter_wait(3)
        if with_deg:
            pltpu.sync_copy(hist, dacc.at[idn], add=True)
        plsc.subcore_barrier()

        pltpu.sync_copy(acc.at[pl.ds(s * zr, zr)], agg_out.at[c, pl.ds(s * zr, zr)])
        if with_deg:
            @pl.when(s < rows // 8)
            def _():
                pltpu.sync_copy(dacc.at[pl.ds(s * 8, 8)],
                                deg_out.at[c, pl.ds(s * 8, 8)])

    return pl.kernel(body, out_type=tuple(out_type), mesh=mesh,
                     scratch_types=scratch,
                     compiler_params=pltpu.CompilerParams(
                         needs_layout_passes=False))


def _dense_body(h_ref, agg_ref, deg_ref, ws_ref, wn_ref, b_ref, out_ref):
    deg = jnp.maximum(deg_ref[...], 1.0)
    agg = (agg_ref[0] + agg_ref[1]) / deg
    hn = jnp.dot(agg, wn_ref[...], preferred_element_type=jnp.float32)
    hs = jnp.dot(h_ref[...], ws_ref[...], preferred_element_type=jnp.float32)
    h = jnp.maximum(hs + hn + b_ref[...], 0.0)
    nrm = jnp.maximum(jnp.sqrt(jnp.sum(h * h, axis=1, keepdims=True)), 1e-12)
    out_ref[...] = h / nrm


def _dense_layer(h, agg, deg, w_self, w_neigh, bias_row, br=400):
    n, d = h.shape
    grid = (n // br,)
    return pl.pallas_call(
        _dense_body,
        grid=grid,
        in_specs=[
            pl.BlockSpec((br, d), lambda i: (i, 0)),
            pl.BlockSpec((NC, br, d), lambda i: (0, i, 0)),
            pl.BlockSpec((br, 1), lambda i: (i, 0)),
            pl.BlockSpec((d, d), lambda i: (0, 0)),
            pl.BlockSpec((d, d), lambda i: (0, 0)),
            pl.BlockSpec((1, d), lambda i: (0, 0)),
        ],
        out_specs=pl.BlockSpec((br, d), lambda i: (i, 0)),
        out_shape=jax.ShapeDtypeStruct((n, d), jnp.float32),
    )(h, agg, deg, w_self, w_neigh, bias_row)


def kernel(x, edge_index, W_self, W_neigh, bias):
    n, d = x.shape
    e = edge_index.shape[1]
    L = W_self.shape[0]

    k = -(-e // (NW * C))
    k += (-k) % 4
    e_pad = k * NW * C
    n_acc = -(-(n + 128) // 128) * 128
    npad_rows = n_acc - n

    src = edge_index[0]
    dst = edge_index[1]
    pad = e_pad - e
    if pad:
        ar = jnp.arange(pad, dtype=jnp.int32)
        src = jnp.concatenate([src, ar % n])
        dst = jnp.concatenate([dst, n + (ar % npad_rows)])
    sd = jnp.stack([src.reshape(NW, k, C), dst.reshape(NW, k, C)], axis=2)

    agg_deg_fn = _sc_agg_kernel(n, d, k, n_acc, with_deg=True)
    agg_fn = _sc_agg_kernel(n, d, k, n_acc)

    agg, degp = agg_deg_fn(x, sd)
    deg = (degp[0] + degp[1]).reshape(n_acc)[:n, None]
    h = _dense_layer(x, agg, deg, W_self[0], W_neigh[0], bias[0][None, :])
    for l in range(1, L):
        (agg,) = agg_fn(h, sd)
        h = _dense_layer(h, agg, deg, W_self[l], W_neigh[l], bias[l][None, :])
    return h


# speedup vs baseline: 1.1067x; 1.0085x over previous
import functools

import jax
import jax.numpy as jnp
from jax import lax
from jax.experimental import pallas as pl
from jax.experimental.pallas import tpu as pltpu
from jax.experimental.pallas import tpu_sc as plsc

NC = 2
NS = 16
NW = NC * NS
C = 128


def _sc_agg_kernel(n, d, k, n_acc, with_deg=False):
    zr = n_acc // NS
    rows = n_acc // 128

    out_type = [jax.ShapeDtypeStruct((NC, n_acc, d), jnp.float32)]
    scratch = [
        pltpu.VMEM_SHARED((n_acc, d), jnp.float32),
        [pltpu.VMEM((2, C), jnp.int32) for _ in range(4)],
        [pltpu.VMEM((C, d), jnp.float32) for _ in range(2)],
        [pltpu.SemaphoreType.DMA for _ in range(2)],
        [pltpu.SemaphoreType.DMA for _ in range(4)],
        [pltpu.SemaphoreType.DMA for _ in range(4)],
    ]
    if with_deg:
        out_type.append(jax.ShapeDtypeStruct((NC, rows, 128), jnp.float32))
        scratch += [
            pltpu.VMEM_SHARED((rows, 128), jnp.float32),
            pltpu.VMEM((rows, 128), jnp.float32),
            pltpu.VMEM((rows,), jnp.int32),
        ]

    mesh = plsc.VectorSubcoreMesh(core_axis_name="c", subcore_axis_name="s",
                                  num_cores=NC, num_subcores=NS)

    def body(*refs):
        if with_deg:
            (h_hbm, sd_hbm, agg_out, deg_out,
             acc, ib, rb, semg, semi, sems, dacc, hist, idn) = refs
        else:
            (h_hbm, sd_hbm, agg_out,
             acc, ib, rb, semg, semi, sems) = refs
        c = lax.axis_index("c")
        s = lax.axis_index("s")
        wid = c * NS + s

        z16 = jnp.zeros((16,), jnp.float32)
        ones = jnp.ones((16,), jnp.float32)

        def count(ibuf):
            for r in range(C // 16):
                v = ibuf[1, pl.ds(r * 16, 16)]
                vhi = lax.shift_right_logical(v, 7)
                vlo = lax.bitwise_and(v, 127)
                plsc.addupdate_scatter(hist, [vhi, vlo], ones)

        def idx_start(j, t):
            pltpu.async_copy(sd_hbm.at[wid, j], ib[t], semi[t])

        def idx_wait(t):
            pltpu.make_async_copy(sd_hbm.at[wid, 0], ib[t], semi[t]).wait()

        def gather_start(ti, tr):
            pltpu.async_copy(h_hbm.at[ib[ti].at[0]], rb[tr], semg[tr])

        def gather_wait(tr):
            pltpu.make_async_copy(h_hbm.at[ib[0].at[0]], rb[tr], semg[tr]).wait()

        def scatter_start(ti, tr):
            pltpu.async_copy(rb[tr], acc.at[ib[ti].at[1]], sems[ti], add=True)

        def scatter_wait(ti):
            pltpu.make_async_copy(rb[ti % 2], acc.at[ib[ti].at[1]],
                                  sems[ti]).wait()

        pltpu.sync_copy(sd_hbm.at[wid, 0], ib[0])
        gather_start(0, 0)
        idx_start(1, 1)

        def zrb(r, carry):
            for cc in range(d // 16):
                rb[1][r, pl.ds(cc * 16, 16)] = z16
            return carry

        lax.fori_loop(0, C, zrb, 0)
        for q in range(zr // C):
            pltpu.sync_copy(rb[1], acc.at[pl.ds(s * zr + q * C, C)])
        if with_deg:
            def zrow(r, carry):
                for cc in range(8):
                    hist[r, pl.ds(cc * 16, 16)] = z16
                return carry

            lax.fori_loop(0, rows, zrow, 0)

            def irow(r, carry):
                idn[pl.ds(r * 16, 16)] = lax.iota(jnp.int32, 16) + r * 16
                return carry

            lax.fori_loop(0, rows // 16, irow, 0)

            @pl.when(s == 0)
            def _():
                pltpu.sync_copy(hist, dacc)
        plsc.subcore_barrier()


        def step(qq, carry):
            j0 = qq * 4
            first = qq == 0
            more = j0 + 4 < k
            for t in range(4):
                j = j0 + t
                tr = t % 2
                gather_wait(tr)
                scatter_start(t, tr)
                if with_deg:
                    count(ib[t])
                if t == 0:
                    @pl.when(jnp.logical_not(first))
                    def _():
                        scatter_wait(3)

                    idx_wait(1)
                    gather_start(1, 1)
                elif t == 3:
                    @pl.when(more)
                    def _():
                        scatter_wait(2)
                        idx_wait(0)
                        gather_start(0, 0)
                else:
                    scatter_wait(t - 1)
                    idx_wait(t + 1)
                    gather_start(t + 1, (t + 1) % 2)
                ti2 = (t + 2) % 4
                if t < 2:
                    idx_start(j + 2, ti2)
                else:
                    @pl.when(more)
                    def _():
                        idx_start(j + 2, ti2)
            return carry

        lax.fori_loop(0, k // 4, step, 0)
        scatter_wait(2)
        scatter_wait(3)
        if with_deg:
            pltpu.sync_copy(hist, dacc.at[idn], add=True)
        plsc.subcore_barrier()

        pltpu.sync_copy(acc.at[pl.ds(s * zr, zr)], agg_out.at[c, pl.ds(s * zr, zr)])
        if with_deg:
            @pl.when(s < rows // 8)
            def _():
                pltpu.sync_copy(dacc.at[pl.ds(s * 8, 8)],
                                deg_out.at[c, pl.ds(s * 8, 8)])

    return pl.kernel(body, out_type=tuple(out_type), mesh=mesh,
                     scratch_types=scratch,
                     compiler_params=pltpu.CompilerParams(
                         needs_layout_passes=False))


def _dense_body(h_ref, agg_ref, deg_ref, ws_ref, wn_ref, b_ref, out_ref):
    deg = jnp.maximum(deg_ref[...], 1.0)
    agg = (agg_ref[0] + agg_ref[1]) / deg
    hn = jnp.dot(agg, wn_ref[...], preferred_element_type=jnp.float32)
    hs = jnp.dot(h_ref[...], ws_ref[...], preferred_element_type=jnp.float32)
    h = jnp.maximum(hs + hn + b_ref[...], 0.0)
    nrm = jnp.maximum(jnp.sqrt(jnp.sum(h * h, axis=1, keepdims=True)), 1e-12)
    out_ref[...] = h / nrm


def _dense_layer(h, agg, deg, w_self, w_neigh, bias_row, br=400):
    n, d = h.shape
    grid = (n // br,)
    return pl.pallas_call(
        _dense_body,
        grid=grid,
        in_specs=[
            pl.BlockSpec((br, d), lambda i: (i, 0)),
            pl.BlockSpec((NC, br, d), lambda i: (0, i, 0)),
            pl.BlockSpec((br, 1), lambda i: (i, 0)),
            pl.BlockSpec((d, d), lambda i: (0, 0)),
            pl.BlockSpec((d, d), lambda i: (0, 0)),
            pl.BlockSpec((1, d), lambda i: (0, 0)),
        ],
        out_specs=pl.BlockSpec((br, d), lambda i: (i, 0)),
        out_shape=jax.ShapeDtypeStruct((n, d), jnp.float32),
    )(h, agg, deg, w_self, w_neigh, bias_row)


def kernel(x, edge_index, W_self, W_neigh, bias):
    n, d = x.shape
    e = edge_index.shape[1]
    L = W_self.shape[0]

    k = -(-e // (NW * C))
    k += (-k) % 4
    e_pad = k * NW * C
    n_acc = -(-(n + 128) // 128) * 128
    npad_rows = n_acc - n

    src = edge_index[0]
    dst = edge_index[1]
    pad = e_pad - e
    if pad:
        ar = jnp.arange(pad, dtype=jnp.int32)
        src = jnp.concatenate([src, ar % n])
        dst = jnp.concatenate([dst, n + (ar % npad_rows)])
    sd = jnp.stack([src.reshape(NW, k, C), dst.reshape(NW, k, C)], axis=2)

    agg_deg_fn = _sc_agg_kernel(n, d, k, n_acc, with_deg=True)
    agg_fn = _sc_agg_kernel(n, d, k, n_acc)

    agg, degp = agg_deg_fn(x, sd)
    deg = (degp[0] + degp[1]).reshape(n_acc)[:n, None]
    h = _dense_layer(x, agg, deg, W_self[0], W_neigh[0], bias[0][None, :])
    for l in range(1, L):
        (agg,) = agg_fn(h, sd)
        h = _dense_layer(h, agg, deg, W_self[l], W_neigh[l], bias[l][None, :])
    return h


# speedup vs baseline: 1.1650x; 1.0527x over previous
import functools

import jax
import jax.numpy as jnp
from jax import lax
from jax.experimental import pallas as pl
from jax.experimental.pallas import tpu as pltpu
from jax.experimental.pallas import tpu_sc as plsc

NC = 2
NS = 16
NW = NC * NS
C = 128


def _sc_agg_kernel(n, d, k, n_acc, with_deg=False):
    zr = n_acc // NS
    rows = n_acc // 128

    out_type = [jax.ShapeDtypeStruct((NC, n_acc, d), jnp.float32)]
    scratch = [
        pltpu.VMEM_SHARED((n_acc, d), jnp.float32),
        [pltpu.VMEM((2, C), jnp.int32) for _ in range(4)],
        [pltpu.VMEM((C, d), jnp.float32) for _ in range(2)],
        [pltpu.SemaphoreType.DMA for _ in range(2)],
        [pltpu.SemaphoreType.DMA for _ in range(4)],
        [pltpu.SemaphoreType.DMA for _ in range(4)],
    ]
    if with_deg:
        out_type.append(jax.ShapeDtypeStruct((NC, rows, 128), jnp.float32))
        scratch += [
            pltpu.VMEM_SHARED((rows, 128), jnp.float32),
            pltpu.VMEM((rows, 128), jnp.float32),
            pltpu.VMEM((rows,), jnp.int32),
        ]

    mesh = plsc.VectorSubcoreMesh(core_axis_name="c", subcore_axis_name="s",
                                  num_cores=NC, num_subcores=NS)

    def body(*refs):
        if with_deg:
            (h_hbm, sd_hbm, agg_out, deg_out,
             acc, ib, rb, semg, semi, sems, dacc, hist, idn) = refs
        else:
            (h_hbm, sd_hbm, agg_out,
             acc, ib, rb, semg, semi, sems) = refs
        c = lax.axis_index("c")
        s = lax.axis_index("s")
        wid = c * NS + s

        z16 = jnp.zeros((16,), jnp.float32)
        ones = jnp.ones((16,), jnp.float32)

        def count(ibuf):
            for r in range(C // 16):
                v = ibuf[1, pl.ds(r * 16, 16)]
                vhi = lax.shift_right_logical(v, 7)
                vlo = lax.bitwise_and(v, 127)
                plsc.addupdate_scatter(hist, [vhi, vlo], ones)

        def idx_start(j, t):
            pltpu.async_copy(sd_hbm.at[wid, j], ib[t], semi[t])

        def idx_wait(t):
            pltpu.make_async_copy(sd_hbm.at[wid, 0], ib[t], semi[t]).wait()

        def gather_start(ti, tr):
            pltpu.async_copy(h_hbm.at[ib[ti].at[0]], rb[tr], semg[tr])

        def gather_wait(tr):
            pltpu.make_async_copy(h_hbm.at[ib[0].at[0]], rb[tr], semg[tr]).wait()

        def scatter_start(ti, tr):
            pltpu.async_copy(rb[tr], acc.at[ib[ti].at[1]], sems[ti], add=True)

        def scatter_wait(ti):
            pltpu.make_async_copy(rb[ti % 2], acc.at[ib[ti].at[1]],
                                  sems[ti]).wait()

        pltpu.sync_copy(sd_hbm.at[wid, 0], ib[0])
        gather_start(0, 0)
        idx_start(1, 1)

        def zrb(r, carry):
            for cc in range(d // 16):
                rb[1][r, pl.ds(cc * 16, 16)] = z16
            return carry

        lax.fori_loop(0, C, zrb, 0)
        for q in range(zr // C):
            pltpu.sync_copy(rb[1], acc.at[pl.ds(s * zr + q * C, C)])
        if with_deg:
            def zrow(r, carry):
                for cc in range(8):
                    hist[r, pl.ds(cc * 16, 16)] = z16
                return carry

            lax.fori_loop(0, rows, zrow, 0)

            def irow(r, carry):
                idn[pl.ds(r * 16, 16)] = lax.iota(jnp.int32, 16) + r * 16
                return carry

            lax.fori_loop(0, rows // 16, irow, 0)

            @pl.when(s == 0)
            def _():
                pltpu.sync_copy(hist, dacc)
        plsc.subcore_barrier()


        def step(qq, carry):
            j0 = qq * 4
            first = qq == 0
            more = j0 + 4 < k
            for t in range(4):
                j = j0 + t
                tr = t % 2
                gather_wait(tr)
                scatter_start(t, tr)
                if with_deg:
                    count(ib[t])
                if t == 0:
                    @pl.when(jnp.logical_not(first))
                    def _():
                        scatter_wait(3)

                    idx_wait(1)
                    gather_start(1, 1)
                elif t == 3:
                    @pl.when(more)
                    def _():
                        scatter_wait(2)
                        idx_wait(0)
                        gather_start(0, 0)
                else:
                    scatter_wait(t - 1)
                    idx_wait(t + 1)
                    gather_start(t + 1, (t + 1) % 2)
                ti2 = (t + 2) % 4
                if t < 2:
                    idx_start(j + 2, ti2)
                else:
                    @pl.when(more)
                    def _():
                        idx_start(j + 2, ti2)
            return carry

        lax.fori_loop(0, k // 4, step, 0)
        scatter_wait(2)
        scatter_wait(3)
        if with_deg:
            pltpu.sync_copy(hist, dacc.at[idn], add=True)
        plsc.subcore_barrier()

        pltpu.sync_copy(acc.at[pl.ds(s * zr, zr)], agg_out.at[c, pl.ds(s * zr, zr)])
        if with_deg:
            @pl.when(s < rows // 8)
            def _():
                pltpu.sync_copy(dacc.at[pl.ds(s * 8, 8)],
                                deg_out.at[c, pl.ds(s * 8, 8)])

    return pl.kernel(body, out_type=tuple(out_type), mesh=mesh,
                     scratch_types=scratch,
                     compiler_params=pltpu.CompilerParams(
                         needs_layout_passes=False))


def _dense_body(h_ref, agg_ref, deg_ref, ws_ref, wn_ref, b_ref, out_ref):
    deg = jnp.maximum(deg_ref[...], 1.0)
    agg = (agg_ref[0] + agg_ref[1]) / deg
    hn = jnp.dot(agg, wn_ref[...], preferred_element_type=jnp.float32)
    hs = jnp.dot(h_ref[...], ws_ref[...], preferred_element_type=jnp.float32)
    h = jnp.maximum(hs + hn + b_ref[...], 0.0)
    nrm = jnp.maximum(jnp.sqrt(jnp.sum(h * h, axis=1, keepdims=True)), 1e-12)
    out_ref[...] = h / nrm


def _dense_layer(h, agg, deg, w_self, w_neigh, bias_row, br=1000):
    n, d = h.shape
    grid = (n // br,)
    return pl.pallas_call(
        _dense_body,
        grid=grid,
        in_specs=[
            pl.BlockSpec((br, d), lambda i: (i, 0)),
            pl.BlockSpec((NC, br, d), lambda i: (0, i, 0)),
            pl.BlockSpec((br, 1), lambda i: (i, 0)),
            pl.BlockSpec((d, d), lambda i: (0, 0)),
            pl.BlockSpec((d, d), lambda i: (0, 0)),
            pl.BlockSpec((1, d), lambda i: (0, 0)),
        ],
        out_specs=pl.BlockSpec((br, d), lambda i: (i, 0)),
        out_shape=jax.ShapeDtypeStruct((n, d), jnp.float32),
    )(h, agg, deg, w_self, w_neigh, bias_row)


def kernel(x, edge_index, W_self, W_neigh, bias):
    n, d = x.shape
    e = edge_index.shape[1]
    L = W_self.shape[0]

    k = -(-e // (NW * C))
    k += (-k) % 4
    e_pad = k * NW * C
    n_acc = -(-(n + 128) // 128) * 128
    npad_rows = n_acc - n

    src = edge_index[0]
    dst = edge_index[1]
    pad = e_pad - e
    if pad:
        ar = jnp.arange(pad, dtype=jnp.int32)
        src = jnp.concatenate([src, ar % n])
        dst = jnp.concatenate([dst, n + (ar % npad_rows)])
    sd = jnp.stack([src.reshape(NW, k, C), dst.reshape(NW, k, C)], axis=2)

    agg_deg_fn = _sc_agg_kernel(n, d, k, n_acc, with_deg=True)
    agg_fn = _sc_agg_kernel(n, d, k, n_acc)

    agg, degp = agg_deg_fn(x, sd)
    deg = (degp[0] + degp[1]).reshape(n_acc)[:n, None]
    h = _dense_layer(x, agg, deg, W_self[0], W_neigh[0], bias[0][None, :])
    for l in range(1, L):
        (agg,) = agg_fn(h, sd)
        h = _dense_layer(h, agg, deg, W_self[l], W_neigh[l], bias[l][None, :])
    return h


# speedup vs baseline: 1.1812x; 1.0139x over previous
import functools

import jax
import jax.numpy as jnp
from jax import lax
from jax.experimental import pallas as pl
from jax.experimental.pallas import tpu as pltpu
from jax.experimental.pallas import tpu_sc as plsc

NC = 2
NS = 16
NW = NC * NS
C = 128


def _sc_agg_kernel(n, d, k, n_acc, with_deg=False):
    zr = n_acc // NS
    rows = n_acc // 128

    out_type = [jax.ShapeDtypeStruct((NC, n_acc, d), jnp.float32)]
    scratch = [
        pltpu.VMEM_SHARED((n_acc, d), jnp.float32),
        [pltpu.VMEM((2, C), jnp.int32) for _ in range(4)],
        [pltpu.VMEM((C, d), jnp.float32) for _ in range(2)],
        [pltpu.SemaphoreType.DMA for _ in range(2)],
        [pltpu.SemaphoreType.DMA for _ in range(4)],
        [pltpu.SemaphoreType.DMA for _ in range(4)],
    ]
    if with_deg:
        out_type.append(jax.ShapeDtypeStruct((NC, rows, 128), jnp.float32))
        scratch += [
            pltpu.VMEM_SHARED((rows, 128), jnp.float32),
            pltpu.VMEM((rows, 128), jnp.float32),
            pltpu.VMEM((rows,), jnp.int32),
        ]

    mesh = plsc.VectorSubcoreMesh(core_axis_name="c", subcore_axis_name="s",
                                  num_cores=NC, num_subcores=NS)

    def body(*refs):
        if with_deg:
            (h_hbm, sd_hbm, agg_out, deg_out,
             acc, ib, rb, semg, semi, sems, dacc, hist, idn) = refs
        else:
            (h_hbm, sd_hbm, agg_out,
             acc, ib, rb, semg, semi, sems) = refs
        c = lax.axis_index("c")
        s = lax.axis_index("s")
        wid = c * NS + s

        z16 = jnp.zeros((16,), jnp.float32)
        ones = jnp.ones((16,), jnp.float32)

        def count(ibuf):
            for r in range(C // 16):
                v = ibuf[1, pl.ds(r * 16, 16)]
                vhi = lax.shift_right_logical(v, 7)
                vlo = lax.bitwise_and(v, 127)
                plsc.addupdate_scatter(hist, [vhi, vlo], ones)

        def idx_start(j, t):
            pltpu.async_copy(sd_hbm.at[wid, j], ib[t], semi[t])

        def idx_wait(t):
            pltpu.make_async_copy(sd_hbm.at[wid, 0], ib[t], semi[t]).wait()

        def gather_start(ti, tr):
            pltpu.async_copy(h_hbm.at[ib[ti].at[0]], rb[tr], semg[tr])

        def gather_wait(tr):
            pltpu.make_async_copy(h_hbm.at[ib[0].at[0]], rb[tr], semg[tr]).wait()

        def scatter_start(ti, tr):
            pltpu.async_copy(rb[tr], acc.at[ib[ti].at[1]], sems[ti], add=True)

        def scatter_wait(ti):
            pltpu.make_async_copy(rb[ti % 2], acc.at[ib[ti].at[1]],
                                  sems[ti]).wait()

        pltpu.sync_copy(sd_hbm.at[wid, 0], ib[0])
        gather_start(0, 0)
        idx_start(1, 1)

        def zrb(r, carry):
            for cc in range(d // 16):
                rb[1][r, pl.ds(cc * 16, 16)] = z16
            return carry

        lax.fori_loop(0, C, zrb, 0)
        for q in range(zr // C):
            pltpu.sync_copy(rb[1], acc.at[pl.ds(s * zr + q * C, C)])
        if with_deg:
            def zrow(r, carry):
                for cc in range(8):
                    hist[r, pl.ds(cc * 16, 16)] = z16
                return carry

            lax.fori_loop(0, rows, zrow, 0)

            def irow(r, carry):
                idn[pl.ds(r * 16, 16)] = lax.iota(jnp.int32, 16) + r * 16
                return carry

            lax.fori_loop(0, rows // 16, irow, 0)

            @pl.when(s == 0)
            def _():
                pltpu.sync_copy(hist, dacc)
        plsc.subcore_barrier()


        def step(qq, carry):
            j0 = qq * 4
            first = qq == 0
            more = j0 + 4 < k
            for t in range(4):
                j = j0 + t
                tr = t % 2
                gather_wait(tr)
                scatter_start(t, tr)
                if with_deg:
                    count(ib[t])
                if t == 0:
                    @pl.when(jnp.logical_not(first))
                    def _():
                        scatter_wait(3)

                    idx_wait(1)
                    gather_start(1, 1)
                elif t == 3:
                    @pl.when(more)
                    def _():
                        scatter_wait(2)
                        idx_wait(0)
                        gather_start(0, 0)
                else:
                    scatter_wait(t - 1)
                    idx_wait(t + 1)
                    gather_start(t + 1, (t + 1) % 2)
                ti2 = (t + 2) % 4
                if t < 2:
                    idx_start(j + 2, ti2)
                else:
                    @pl.when(more)
                    def _():
                        idx_start(j + 2, ti2)
            return carry

        lax.fori_loop(0, k // 4, step, 0)
        scatter_wait(2)
        scatter_wait(3)
        if with_deg:
            pltpu.sync_copy(hist, dacc.at[idn], add=True)
        plsc.subcore_barrier()

        pltpu.sync_copy(acc.at[pl.ds(s * zr, zr)], agg_out.at[c, pl.ds(s * zr, zr)])
        if with_deg:
            @pl.when(s < rows // 8)
            def _():
                pltpu.sync_copy(dacc.at[pl.ds(s * 8, 8)],
                                deg_out.at[c, pl.ds(s * 8, 8)])

    return pl.kernel(body, out_type=tuple(out_type), mesh=mesh,
                     scratch_types=scratch,
                     compiler_params=pltpu.CompilerParams(
                         needs_layout_passes=False))


def _dense_body(h_ref, agg_ref, deg_ref, ws_ref, wn_ref, b_ref, out_ref):
    deg = jnp.maximum(deg_ref[...], 1.0)
    agg = (agg_ref[0] + agg_ref[1]) / deg
    hn = jnp.dot(agg, wn_ref[...], preferred_element_type=jnp.float32)
    hs = jnp.dot(h_ref[...], ws_ref[...], preferred_element_type=jnp.float32)
    h = jnp.maximum(hs + hn + b_ref[...], 0.0)
    nrm = jnp.maximum(jnp.sqrt(jnp.sum(h * h, axis=1, keepdims=True)), 1e-12)
    out_ref[...] = h / nrm


def _dense_layer(h, agg, deg, w_self, w_neigh, bias_row, br=2000):
    n, d = h.shape
    grid = (n // br,)
    return pl.pallas_call(
        _dense_body,
        grid=grid,
        in_specs=[
            pl.BlockSpec((br, d), lambda i: (i, 0)),
            pl.BlockSpec((NC, br, d), lambda i: (0, i, 0)),
            pl.BlockSpec((br, 1), lambda i: (i, 0)),
            pl.BlockSpec((d, d), lambda i: (0, 0)),
            pl.BlockSpec((d, d), lambda i: (0, 0)),
            pl.BlockSpec((1, d), lambda i: (0, 0)),
        ],
        out_specs=pl.BlockSpec((br, d), lambda i: (i, 0)),
        out_shape=jax.ShapeDtypeStruct((n, d), jnp.float32),
    )(h, agg, deg, w_self, w_neigh, bias_row)


def kernel(x, edge_index, W_self, W_neigh, bias):
    n, d = x.shape
    e = edge_index.shape[1]
    L = W_self.shape[0]

    k = -(-e // (NW * C))
    k += (-k) % 4
    e_pad = k * NW * C
    n_acc = -(-(n + 128) // 128) * 128
    npad_rows = n_acc - n

    src = edge_index[0]
    dst = edge_index[1]
    pad = e_pad - e
    if pad:
        ar = jnp.arange(pad, dtype=jnp.int32)
        src = jnp.concatenate([src, ar % n])
        dst = jnp.concatenate([dst, n + (ar % npad_rows)])
    sd = jnp.stack([src.reshape(NW, k, C), dst.reshape(NW, k, C)], axis=2)

    agg_deg_fn = _sc_agg_kernel(n, d, k, n_acc, with_deg=True)
    agg_fn = _sc_agg_kernel(n, d, k, n_acc)

    agg, degp = agg_deg_fn(x, sd)
    deg = (degp[0] + degp[1]).reshape(n_acc)[:n, None]
    h = _dense_layer(x, agg, deg, W_self[0], W_neigh[0], bias[0][None, :])
    for l in range(1, L):
        (agg,) = agg_fn(h, sd)
        h = _dense_layer(h, agg, deg, W_self[l], W_neigh[l], bias[l][None, :])
    return h


# speedup vs baseline: 1.1830x; 1.0015x over previous
import jax
import jax.numpy as jnp
from jax import lax
from jax.experimental import pallas as pl
from jax.experimental.pallas import tpu as pltpu
from jax.experimental.pallas import tpu_sc as plsc

NC = 2
NS = 16
NW = NC * NS
C = 128


def _sc_agg_kernel(n, d, k, n_acc, with_deg=False):
    zr = n_acc // NS
    rows = n_acc // 128

    out_type = [jax.ShapeDtypeStruct((NC, n_acc, d), jnp.float32)]
    scratch = [
        pltpu.VMEM_SHARED((n_acc, d), jnp.float32),
        [pltpu.VMEM((2, C), jnp.int32) for _ in range(4)],
        [pltpu.VMEM((C, d), jnp.float32) for _ in range(2)],
        [pltpu.SemaphoreType.DMA for _ in range(2)],
        [pltpu.SemaphoreType.DMA for _ in range(4)],
        [pltpu.SemaphoreType.DMA for _ in range(4)],
    ]
    if with_deg:
        out_type.append(jax.ShapeDtypeStruct((NC, rows, 128), jnp.float32))
        scratch += [
            pltpu.VMEM_SHARED((rows, 128), jnp.float32),
            pltpu.VMEM((rows, 128), jnp.float32),
            pltpu.VMEM((rows,), jnp.int32),
        ]

    mesh = plsc.VectorSubcoreMesh(core_axis_name="c", subcore_axis_name="s",
                                  num_cores=NC, num_subcores=NS)

    def body(*refs):
        if with_deg:
            (h_hbm, sd_hbm, agg_out, deg_out,
             acc, ib, rb, semg, semi, sems, dacc, hist, idn) = refs
        else:
            (h_hbm, sd_hbm, agg_out,
             acc, ib, rb, semg, semi, sems) = refs
        c = lax.axis_index("c")
        s = lax.axis_index("s")
        wid = c * NS + s

        z16 = jnp.zeros((16,), jnp.float32)
        ones = jnp.ones((16,), jnp.float32)

        def count(ibuf):
            for r in range(C // 16):
                v = ibuf[1, pl.ds(r * 16, 16)]
                vhi = lax.shift_right_logical(v, 7)
                vlo = lax.bitwise_and(v, 127)
                plsc.addupdate_scatter(hist, [vhi, vlo], ones)

        def idx_start(j, t):
            pltpu.async_copy(sd_hbm.at[wid, j], ib[t], semi[t])

        def idx_wait(t):
            pltpu.make_async_copy(sd_hbm.at[wid, 0], ib[t], semi[t]).wait()

        def gather_start(ti, tr):
            pltpu.async_copy(h_hbm.at[ib[ti].at[0]], rb[tr], semg[tr])

        def gather_wait(tr):
            pltpu.make_async_copy(h_hbm.at[ib[0].at[0]], rb[tr], semg[tr]).wait()

        def scatter_start(ti, tr):
            pltpu.async_copy(rb[tr], acc.at[ib[ti].at[1]], sems[ti], add=True)

        def scatter_wait(ti):
            pltpu.make_async_copy(rb[ti % 2], acc.at[ib[ti].at[1]],
                                  sems[ti]).wait()

        pltpu.sync_copy(sd_hbm.at[wid, 0], ib[0])
        gather_start(0, 0)
        idx_start(1, 1)

        def zrb(r, carry):
            for cc in range(d // 16):
                rb[1][r, pl.ds(cc * 16, 16)] = z16
            return carry

        lax.fori_loop(0, C, zrb, 0)
        for q in range(zr // C):
            pltpu.sync_copy(rb[1], acc.at[pl.ds(s * zr + q * C, C)])
        if with_deg:
            def zrow(r, carry):
                for cc in range(8):
                    hist[r, pl.ds(cc * 16, 16)] = z16
                return carry

            lax.fori_loop(0, rows, zrow, 0)

            def irow(r, carry):
                idn[pl.ds(r * 16, 16)] = lax.iota(jnp.int32, 16) + r * 16
                return carry

            lax.fori_loop(0, rows // 16, irow, 0)

            @pl.when(s == 0)
            def _():
                pltpu.sync_copy(hist, dacc)
        plsc.subcore_barrier()


        def step(qq, carry):
            j0 = qq * 4
            first = qq == 0
            more = j0 + 4 < k
            for t in range(4):
                j = j0 + t
                tr = t % 2
                gather_wait(tr)
                scatter_start(t, tr)
                if with_deg:
                    count(ib[t])
                if t == 0:
                    @pl.when(jnp.logical_not(first))
                    def _():
                        scatter_wait(3)

                    idx_wait(1)
                    gather_start(1, 1)
                elif t == 3:
                    @pl.when(more)
                    def _():
                        scatter_wait(2)
                        idx_wait(0)
                        gather_start(0, 0)
                else:
                    scatter_wait(t - 1)
                    idx_wait(t + 1)
                    gather_start(t + 1, (t + 1) % 2)
                ti2 = (t + 2) % 4
                if t < 2:
                    idx_start(j + 2, ti2)
                else:
                    @pl.when(more)
                    def _():
                        idx_start(j + 2, ti2)
            return carry

        lax.fori_loop(0, k // 4, step, 0)
        scatter_wait(2)
        scatter_wait(3)
        if with_deg:
            pltpu.sync_copy(hist, dacc.at[idn], add=True)
        plsc.subcore_barrier()

        pltpu.sync_copy(acc.at[pl.ds(s * zr, zr)], agg_out.at[c, pl.ds(s * zr, zr)])
        if with_deg:
            @pl.when(s < rows // 8)
            def _():
                pltpu.sync_copy(dacc.at[pl.ds(s * 8, 8)],
                                deg_out.at[c, pl.ds(s * 8, 8)])

    return pl.kernel(body, out_type=tuple(out_type), mesh=mesh,
                     scratch_types=scratch,
                     compiler_params=pltpu.CompilerParams(
                         needs_layout_passes=False))


def _dense_body(h_ref, agg_ref, deg_ref, ws_ref, wn_ref, b_ref, out_ref):
    deg = jnp.maximum(deg_ref[...], 1.0)
    agg = (agg_ref[0] + agg_ref[1]) / deg
    hn = jnp.dot(agg, wn_ref[...], preferred_element_type=jnp.float32)
    hs = jnp.dot(h_ref[...], ws_ref[...], preferred_element_type=jnp.float32)
    h = jnp.maximum(hs + hn + b_ref[...], 0.0)
    nrm = jnp.maximum(jnp.sqrt(jnp.sum(h * h, axis=1, keepdims=True)), 1e-12)
    out_ref[...] = h / nrm


def _dense_layer(h, agg, deg, w_self, w_neigh, bias_row, br=2000):
    n, d = h.shape
    grid = (n // br,)
    return pl.pallas_call(
        _dense_body,
        grid=grid,
        in_specs=[
            pl.BlockSpec((br, d), lambda i: (i, 0)),
            pl.BlockSpec((NC, br, d), lambda i: (0, i, 0)),
            pl.BlockSpec((br, 1), lambda i: (i, 0)),
            pl.BlockSpec((d, d), lambda i: (0, 0)),
            pl.BlockSpec((d, d), lambda i: (0, 0)),
            pl.BlockSpec((1, d), lambda i: (0, 0)),
        ],
        out_specs=pl.BlockSpec((br, d), lambda i: (i, 0)),
        out_shape=jax.ShapeDtypeStruct((n, d), jnp.float32),
    )(h, agg, deg, w_self, w_neigh, bias_row)


def kernel(x, edge_index, W_self, W_neigh, bias):
    n, d = x.shape
    e = edge_index.shape[1]
    L = W_self.shape[0]

    k = -(-e // (NW * C))
    k += (-k) % 4
    e_pad = k * NW * C
    n_acc = -(-(n + 128) // 128) * 128
    npad_rows = n_acc - n

    src = edge_index[0]
    dst = edge_index[1]
    pad = e_pad - e
    if pad:
        ar = jnp.arange(pad, dtype=jnp.int32)
        src = jnp.concatenate([src, ar % n])
        dst = jnp.concatenate([dst, n + (ar % npad_rows)])
    sd = jnp.stack([src.reshape(NW, k, C), dst.reshape(NW, k, C)], axis=2)

    agg_deg_fn = _sc_agg_kernel(n, d, k, n_acc, with_deg=True)
    agg_fn = _sc_agg_kernel(n, d, k, n_acc)

    agg, degp = agg_deg_fn(x, sd)
    deg = (degp[0] + degp[1]).reshape(n_acc)[:n, None]
    h = _dense_layer(x, agg, deg, W_self[0], W_neigh[0], bias[0][None, :])
    for l in range(1, L):
        (agg,) = agg_fn(h, sd)
        h = _dense_layer(h, agg, deg, W_self[l], W_neigh[l], bias[l][None, :])
    return h
